# baseline (speedup 1.0000x reference)
import jax
import jax.numpy as jnp
from jax import lax
from jax.experimental import pallas as pl
from jax.experimental.pallas import tpu as pltpu
from jax.experimental.pallas import tpu_sc as plsc

N_TOTAL = 16 * 512 * 512
NROW = N_TOTAL // 512
NW = 32
ROW_W = NROW // NW
RPC = 16
NCHUNK = ROW_W // RPC
VPC = RPC * 512 // 16

NB = 512
HI = 16.0
S16 = float(NB * 16 / HI)
Y16 = float(NB * 16)
CLAMP = float(NB * 16 - 1)
HW = 2 * NB * 16
HR = HW // 128
MASK = 2 * NB * 16 - 1

_f32 = jnp.float32


def _sc_hist_body(logits_hbm, labels_hbm, hist_out, aux_out,
                  lbuf0, lbuf1, ybuf0, ybuf1, cnt, srelu, auxbuf, sem):
    lbufs = (lbuf0, lbuf1)
    ybufs = (ybuf0, ybuf1)
    wid = lax.axis_index("s") * 2 + lax.axis_index("c")
    base_row = wid * ROW_W

    zeros16 = jnp.zeros((16,), _f32)
    ones16 = jnp.ones((16,), _f32)

    def start(c):
        row0 = base_row + c * RPC
        slot = c % 2
        return (
            pltpu.async_copy(logits_hbm.at[pl.ds(row0, RPC), :],
                             lbufs[slot], sem.at[slot]),
            pltpu.async_copy(labels_hbm.at[pl.ds(row0, RPC), :],
                             ybufs[slot], sem.at[slot]),
        )

    pending = {0: start(0), 1: start(1)}

    @plsc.parallel_loop(0, HW // 16, unroll=8)
    def _zero(i):
        cnt[pl.ds(i * 16, 16)] = zeros16
        srelu[pl.ds(i * 16, 16)] = zeros16

    maxv = jnp.zeros((16,), _f32)
    for c in range(NCHUNK):
        if c + 1 < NCHUNK and c + 1 not in pending:
            pending[c + 1] = start(c + 1)
        for h in pending.pop(c):
            h.wait()
        lb_s = lbufs[c % 2]
        yb_s = ybufs[c % 2]

        def vec_body(j, mx, lb_s=lb_s, yb_s=yb_s):
            r = lax.shift_right_logical(j, 5)
            cc = lax.shift_left(j & 31, 4)
            lg = lb_s[r, pl.ds(cc, 16)]
            y = yb_s[r, pl.ds(cc, 16)]
            lgs = lg * S16
            t1 = y + y
            v = t1 - 1.0
            es = S16 - lgs * v
            rs = jnp.maximum(es, 0.0)
            b = jnp.minimum(rs, CLAMP)
            idxf = (t1 * (Y16 * 0.5) + b) + 8388608.0
            bits = plsc.bitcast(idxf, jnp.int32)
            idx = bits & MASK
            plsc.addupdate_scatter(cnt, [idx], ones16)
            plsc.addupdate_scatter(srelu, [idx], rs)
            return jnp.maximum(mx, rs)

        maxv = plsc.parallel_loop(0, VPC, unroll=8, carry=maxv)(vec_body)

    auxbuf[pl.ds(0, 16)] = maxv
    cpy = (pltpu.async_copy(cnt, hist_out.at[pl.ds(wid * 2 * HW, HW)], sem.at[0]),
           pltpu.async_copy(srelu, hist_out.at[pl.ds(wid * 2 * HW + HW, HW)], sem.at[1]),
           pltpu.async_copy(auxbuf, aux_out.at[pl.ds(wid * 16, 16)], sem.at[0]))
    for h in cpy:
        h.wait()


_sc_hist = pl.kernel(
    _sc_hist_body,
    out_type=(
        jax.ShapeDtypeStruct((NW * 2 * HW,), _f32),
        jax.ShapeDtypeStruct((NW * 16,), _f32),
    ),
    mesh=plsc.VectorSubcoreMesh(core_axis_name="c", subcore_axis_name="s"),
    scratch_types=[
        pltpu.VMEM((RPC, 512), _f32),
        pltpu.VMEM((RPC, 512), _f32),
        pltpu.VMEM((RPC, 512), _f32),
        pltpu.VMEM((RPC, 512), _f32),
        pltpu.VMEM((HW,), _f32),
        pltpu.VMEM((HW,), _f32),
        pltpu.VMEM((16,), _f32),
        pltpu.SemaphoreType.DMA((2,)),
    ],
    compiler_params=pltpu.CompilerParams(
        needs_layout_passes=False, use_tc_tiling_on_sc=True),
)


def _tc_reduce_body(hist_ref, aux_ref, out_ref):
    h = hist_ref[...]
    hs = jnp.sum(h, axis=0)
    cn, cp = hs[0, 0:64], hs[0, 64:128]
    sn, sp = hs[1, 0:64], hs[1, 64:128]

    iota_r = lax.broadcasted_iota(jnp.int32, (128, 128), 0)
    iota_c = lax.broadcasted_iota(jnp.int32, (128, 128), 1)
    br = lax.shift_right_logical(iota_r, 4)
    bc = lax.shift_right_logical(iota_c, 4)
    m_incl = (br <= bc).astype(_f32)
    m_bin = (br == bc).astype(_f32)
    ones_m = jnp.ones((128, 128), _f32)
    s_lo = (iota_r[0:64, 0:64] > iota_c[0:64, 0:64]).astype(_f32)

    def mm(a, b):
        return jnp.dot(a, b, preferred_element_type=_f32,
                       precision=lax.Precision.HIGHEST)

    def fp(a):
        return mm(a, m_incl) + mm(s_lo, mm(a, ones_m))

    tot_n = jnp.sum(cn)
    g = jnp.sum(cp)

    nba = tot_n - fp(cn)
    pa = g - fp(cp) + mm(cp, m_bin)
    t = mm(cn, m_bin)

    maxe = jnp.max(aux_ref[...])

    loss_pos = jnp.sum(sp / (g + nba))
    den = (g + nba) * (g + nba + t)
    loss_neg = jnp.sum(jnp.where(t > 0.0, sn * (g - pa) / den, 0.0))
    loss = (loss_pos + loss_neg) * (1.0 / S16)
    out_ref[0, 0] = jnp.where(g > 0.0, loss, jnp.maximum(maxe, 0.0) * (1.0 / S16))


_tc_reduce = pl.pallas_call(
    _tc_reduce_body,
    out_shape=jax.ShapeDtypeStruct((1, 1), _f32),
    out_specs=pl.BlockSpec(memory_space=pltpu.SMEM),
)


def kernel(logits, labels):
    lg = logits.reshape(NROW, 512)
    lb = labels.reshape(NROW, 512)
    hist, aux = _sc_hist(lg, lb)
    hist4 = hist.reshape(NW, 2, HR, 128)
    out = _tc_reduce(hist4, aux.reshape(4, 128))
    return out[0, 0]


# speedup vs baseline: 1.4669x; 1.4669x over previous
import jax
import jax.numpy as jnp
from jax import lax
from jax.experimental import pallas as pl
from jax.experimental.pallas import tpu as pltpu
from jax.experimental.pallas import tpu_sc as plsc

N_TOTAL = 16 * 512 * 512
NROW = N_TOTAL // 512
NW = 32
ROW_W = NROW // NW
RPC = 16
NCHUNK = ROW_W // RPC
VPC = RPC * 512 // 16

NB = 512
HI = 16.0
S16 = float(NB * 16 / HI)
Y16 = float(NB * 16)
CLAMP = float(NB * 16 - 1)
HW = 2 * NB * 16
HR = HW // 128
MASK = 2 * NB * 16 - 16

_f32 = jnp.float32


def _sc_hist_body(logits_hbm, labels_hbm, hist_out, aux_out,
                  lbuf0, lbuf1, ybuf0, ybuf1, cnt, srelu, auxbuf, sem):
    lbufs = (lbuf0, lbuf1)
    ybufs = (ybuf0, ybuf1)
    wid = lax.axis_index("s") * 2 + lax.axis_index("c")
    base_row = wid * ROW_W

    zeros16 = jnp.zeros((16,), _f32)
    ones16 = jnp.ones((16,), _f32)
    lane = lax.broadcasted_iota(jnp.int32, (16,), 0)

    def start(c):
        row0 = base_row + c * RPC
        slot = c % 2
        return (
            pltpu.async_copy(logits_hbm.at[pl.ds(row0, RPC), :],
                             lbufs[slot], sem.at[slot]),
            pltpu.async_copy(labels_hbm.at[pl.ds(row0, RPC), :],
                             ybufs[slot], sem.at[slot]),
        )

    pending = {0: start(0), 1: start(1)}

    @plsc.parallel_loop(0, HW // 16, unroll=8)
    def _zero(i):
        cnt[pl.ds(i * 16, 16)] = zeros16
        srelu[pl.ds(i * 16, 16)] = zeros16

    maxv = jnp.zeros((16,), _f32)
    for c in range(NCHUNK):
        if c + 1 < NCHUNK and c + 1 not in pending:
            pending[c + 1] = start(c + 1)
        for h in pending.pop(c):
            h.wait()
        lb_s = lbufs[c % 2]
        yb_s = ybufs[c % 2]

        def vec_body(j, mx, lb_s=lb_s, yb_s=yb_s):
            r = lax.shift_right_logical(j, 5)
            cc = lax.shift_left(j & 31, 4)
            lg = lb_s[r, pl.ds(cc, 16)]
            y = yb_s[r, pl.ds(cc, 16)]
            lgs = lg * S16
            t1 = y + y
            v = t1 - 1.0
            es = S16 - lgs * v
            rs = jnp.maximum(es, 0.0)
            b = jnp.minimum(rs, CLAMP)
            idxf = (t1 * (Y16 * 0.5) + b) + 8388608.0
            bits = plsc.bitcast(idxf, jnp.int32)
            idx = (bits & MASK) | lane
            plsc.addupdate_scatter(cnt, [idx], ones16)
            plsc.addupdate_scatter(srelu, [idx], rs)
            return jnp.maximum(mx, rs)

        maxv = plsc.parallel_loop(0, VPC, unroll=8, carry=maxv)(vec_body)

    auxbuf[pl.ds(0, 16)] = maxv
    cpy = (pltpu.async_copy(cnt, hist_out.at[pl.ds(wid * 2 * HW, HW)], sem.at[0]),
           pltpu.async_copy(srelu, hist_out.at[pl.ds(wid * 2 * HW + HW, HW)], sem.at[1]),
           pltpu.async_copy(auxbuf, aux_out.at[pl.ds(wid * 16, 16)], sem.at[0]))
    for h in cpy:
        h.wait()


_sc_hist = pl.kernel(
    _sc_hist_body,
    out_type=(
        jax.ShapeDtypeStruct((NW * 2 * HW,), _f32),
        jax.ShapeDtypeStruct((NW * 16,), _f32),
    ),
    mesh=plsc.VectorSubcoreMesh(core_axis_name="c", subcore_axis_name="s"),
    scratch_types=[
        pltpu.VMEM((RPC, 512), _f32),
        pltpu.VMEM((RPC, 512), _f32),
        pltpu.VMEM((RPC, 512), _f32),
        pltpu.VMEM((RPC, 512), _f32),
        pltpu.VMEM((HW,), _f32),
        pltpu.VMEM((HW,), _f32),
        pltpu.VMEM((16,), _f32),
        pltpu.SemaphoreType.DMA((2,)),
    ],
    compiler_params=pltpu.CompilerParams(
        needs_layout_passes=False, use_tc_tiling_on_sc=True),
)


def _tc_reduce_body(hist_ref, aux_ref, out_ref):
    h = hist_ref[...]
    hs = jnp.sum(h, axis=0)
    cn, cp = hs[0, 0:64], hs[0, 64:128]
    sn, sp = hs[1, 0:64], hs[1, 64:128]

    iota_r = lax.broadcasted_iota(jnp.int32, (128, 128), 0)
    iota_c = lax.broadcasted_iota(jnp.int32, (128, 128), 1)
    br = lax.shift_right_logical(iota_r, 4)
    bc = lax.shift_right_logical(iota_c, 4)
    m_incl = (br <= bc).astype(_f32)
    m_bin = (br == bc).astype(_f32)
    ones_m = jnp.ones((128, 128), _f32)
    s_lo = (iota_r[0:64, 0:64] > iota_c[0:64, 0:64]).astype(_f32)

    def mm(a, b):
        return jnp.dot(a, b, preferred_element_type=_f32,
                       precision=lax.Precision.HIGHEST)

    def fp(a):
        return mm(a, m_incl) + mm(s_lo, mm(a, ones_m))

    tot_n = jnp.sum(cn)
    g = jnp.sum(cp)

    nba = tot_n - fp(cn)
    pa = g - fp(cp) + mm(cp, m_bin)
    t = mm(cn, m_bin)

    maxe = jnp.max(aux_ref[...])

    loss_pos = jnp.sum(sp / (g + nba))
    den = (g + nba) * (g + nba + t)
    loss_neg = jnp.sum(jnp.where(t > 0.0, sn * (g - pa) / den, 0.0))
    loss = (loss_pos + loss_neg) * (1.0 / S16)
    out_ref[0, 0] = jnp.where(g > 0.0, loss, jnp.maximum(maxe, 0.0) * (1.0 / S16))


_tc_reduce = pl.pallas_call(
    _tc_reduce_body,
    out_shape=jax.ShapeDtypeStruct((1, 1), _f32),
    out_specs=pl.BlockSpec(memory_space=pltpu.SMEM),
)


def kernel(logits, labels):
    lg = logits.reshape(NROW, 512)
    lb = labels.reshape(NROW, 512)
    hist, aux = _sc_hist(lg, lb)
    hist4 = hist.reshape(NW, 2, HR, 128)
    out = _tc_reduce(hist4, aux.reshape(4, 128))
    return out[0, 0]


# speedup vs baseline: 1.5302x; 1.0431x over previous
import jax
import jax.numpy as jnp
from jax import lax
from jax.experimental import pallas as pl
from jax.experimental.pallas import tpu as pltpu
from jax.experimental.pallas import tpu_sc as plsc

N_TOTAL = 16 * 512 * 512
NROW = N_TOTAL // 512
NW = 32
ROW_W = NROW // NW
RPC = 32
NCHUNK = ROW_W // RPC
VPC = RPC * 512 // 16

NB = 512
HI = 16.0
S16 = float(NB * 16 / HI)
Y16 = float(NB * 16)
CLAMP = float(NB * 16 - 1)
HW = 2 * NB * 16
HR = HW // 128
MASK = 2 * NB * 16 - 16

_f32 = jnp.float32


def _sc_hist_body(logits_hbm, labels_hbm, hist_out, aux_out,
                  lbuf0, lbuf1, ybuf0, ybuf1, cnt, srelu, auxbuf, sem):
    lbufs = (lbuf0, lbuf1)
    ybufs = (ybuf0, ybuf1)
    wid = lax.axis_index("s") * 2 + lax.axis_index("c")
    base_row = wid * ROW_W

    zeros16 = jnp.zeros((16,), _f32)
    ones16 = jnp.ones((16,), _f32)
    lane = lax.broadcasted_iota(jnp.int32, (16,), 0)

    def start(c):
        row0 = base_row + c * RPC
        slot = c % 2
        return (
            pltpu.async_copy(logits_hbm.at[pl.ds(row0, RPC), :],
                             lbufs[slot], sem.at[slot]),
            pltpu.async_copy(labels_hbm.at[pl.ds(row0, RPC), :],
                             ybufs[slot], sem.at[slot]),
        )

    pending = {0: start(0), 1: start(1)}

    @plsc.parallel_loop(0, HW // 16, unroll=8)
    def _zero(i):
        cnt[pl.ds(i * 16, 16)] = zeros16
        srelu[pl.ds(i * 16, 16)] = zeros16

    maxv = jnp.zeros((16,), _f32)
    for c in range(NCHUNK):
        if c + 1 < NCHUNK and c + 1 not in pending:
            pending[c + 1] = start(c + 1)
        for h in pending.pop(c):
            h.wait()
        lb_s = lbufs[c % 2]
        yb_s = ybufs[c % 2]

        def vec_body(j, mx, lb_s=lb_s, yb_s=yb_s):
            r = lax.shift_right_logical(j, 5)
            cc = lax.shift_left(j & 31, 4)
            lg = lb_s[r, pl.ds(cc, 16)]
            y = yb_s[r, pl.ds(cc, 16)]
            lgs = lg * S16
            t1 = y + y
            v = t1 - 1.0
            es = S16 - lgs * v
            rs = jnp.maximum(es, 0.0)
            b = jnp.minimum(rs, CLAMP)
            idxf = (t1 * (Y16 * 0.5) + b) + 8388608.0
            bits = plsc.bitcast(idxf, jnp.int32)
            idx = (bits & MASK) | lane
            plsc.addupdate_scatter(cnt, [idx], ones16)
            plsc.addupdate_scatter(srelu, [idx], rs)
            return jnp.maximum(mx, rs)

        maxv = plsc.parallel_loop(0, VPC, unroll=8, carry=maxv)(vec_body)

    auxbuf[pl.ds(0, 16)] = maxv
    cpy = (pltpu.async_copy(cnt, hist_out.at[pl.ds(wid * 2 * HW, HW)], sem.at[0]),
           pltpu.async_copy(srelu, hist_out.at[pl.ds(wid * 2 * HW + HW, HW)], sem.at[1]),
           pltpu.async_copy(auxbuf, aux_out.at[pl.ds(wid * 16, 16)], sem.at[0]))
    for h in cpy:
        h.wait()


_sc_hist = pl.kernel(
    _sc_hist_body,
    out_type=(
        jax.ShapeDtypeStruct((NW * 2 * HW,), _f32),
        jax.ShapeDtypeStruct((NW * 16,), _f32),
    ),
    mesh=plsc.VectorSubcoreMesh(core_axis_name="c", subcore_axis_name="s"),
    scratch_types=[
        pltpu.VMEM((RPC, 512), _f32),
        pltpu.VMEM((RPC, 512), _f32),
        pltpu.VMEM((RPC, 512), _f32),
        pltpu.VMEM((RPC, 512), _f32),
        pltpu.VMEM((HW,), _f32),
        pltpu.VMEM((HW,), _f32),
        pltpu.VMEM((16,), _f32),
        pltpu.SemaphoreType.DMA((2,)),
    ],
    compiler_params=pltpu.CompilerParams(
        needs_layout_passes=False, use_tc_tiling_on_sc=True),
)


def _tc_reduce_body(hist_ref, aux_ref, out_ref):
    h = hist_ref[...]
    hs = jnp.sum(h, axis=0)
    cn, cp = hs[0, 0:64], hs[0, 64:128]
    sn, sp = hs[1, 0:64], hs[1, 64:128]

    iota_r = lax.broadcasted_iota(jnp.int32, (128, 128), 0)
    iota_c = lax.broadcasted_iota(jnp.int32, (128, 128), 1)
    br = lax.shift_right_logical(iota_r, 4)
    bc = lax.shift_right_logical(iota_c, 4)
    m_incl = (br <= bc).astype(_f32)
    m_bin = (br == bc).astype(_f32)
    ones_m = jnp.ones((128, 128), _f32)
    s_lo = (iota_r[0:64, 0:64] > iota_c[0:64, 0:64]).astype(_f32)

    def mm(a, b):
        return jnp.dot(a, b, preferred_element_type=_f32,
                       precision=lax.Precision.HIGHEST)

    def fp(a):
        return mm(a, m_incl) + mm(s_lo, mm(a, ones_m))

    tot_n = jnp.sum(cn)
    g = jnp.sum(cp)

    nba = tot_n - fp(cn)
    pa = g - fp(cp) + mm(cp, m_bin)
    t = mm(cn, m_bin)

    maxe = jnp.max(aux_ref[...])

    loss_pos = jnp.sum(sp / (g + nba))
    den = (g + nba) * (g + nba + t)
    loss_neg = jnp.sum(jnp.where(t > 0.0, sn * (g - pa) / den, 0.0))
    loss = (loss_pos + loss_neg) * (1.0 / S16)
    out_ref[0, 0] = jnp.where(g > 0.0, loss, jnp.maximum(maxe, 0.0) * (1.0 / S16))


_tc_reduce = pl.pallas_call(
    _tc_reduce_body,
    out_shape=jax.ShapeDtypeStruct((1, 1), _f32),
    out_specs=pl.BlockSpec(memory_space=pltpu.SMEM),
)


def kernel(logits, labels):
    lg = logits.reshape(NROW, 512)
    lb = labels.reshape(NROW, 512)
    hist, aux = _sc_hist(lg, lb)
    hist4 = hist.reshape(NW, 2, HR, 128)
    out = _tc_reduce(hist4, aux.reshape(4, 128))
    return out[0, 0]
